# baseline (speedup 1.0000x reference)
import functools

import jax
import jax.numpy as jnp
from jax import lax
from jax.experimental import pallas as pl
from jax.experimental.pallas import tpu as pltpu
from jax.experimental.pallas import tpu_sc as plsc

EMB = 128
NC = 2
NS = 16
NW = NC * NS
CH = 128
NBUF = 5


def _neg_ring(wid, tbl, out, nidx, nch, bufs, gsems, wsems):
    nb = nch * CH

    def body(g, carry):
        cps = []
        for j in range(NBUF):
            @pl.when(g > 0)
            def _(j=j):
                pltpu.make_async_copy(out.at[pl.ds(wid * nb, CH)],
                                      bufs[j], wsems[j]).wait()
            c = NBUF * g + j
            cps.append(pltpu.async_copy(tbl.at[nidx.at[c]], bufs[j],
                                        gsems[j]))
        for j in range(NBUF):
            cps[j].wait()
            c = NBUF * g + j
            pltpu.async_copy(bufs[j], out.at[pl.ds(wid * nb + c * CH, CH)],
                             wsems[j])
        return carry

    lax.fori_loop(0, nch // NBUF, body, 0)
    for j in range(NBUF):
        pltpu.make_async_copy(out.at[pl.ds(wid * nb, CH)], bufs[j],
                              wsems[j]).wait()


def _sc_gather_all(users, items, neg_chunks, U_mlp, U_mf, U_mlp_g, U_mf_g,
                   V_mlp, V_mf):
    B = users.shape[0]
    nch = neg_chunks.shape[1]
    NB = NW * nch * CH
    ub = B // NW

    mesh = plsc.VectorSubcoreMesh(core_axis_name="c", subcore_axis_name="s")
    f32 = jnp.float32
    out_type = (
        [jax.ShapeDtypeStruct((B, EMB), f32)] * 6
        + [jax.ShapeDtypeStruct((NB, EMB), f32)] * 2
    )
    scratch_types = (
        [pltpu.VMEM((ub,), jnp.int32),
         pltpu.VMEM((ub,), jnp.int32),
         pltpu.VMEM((nch, CH), jnp.int32),
         pltpu.VMEM((ub, EMB), f32),
         pltpu.VMEM((ub, EMB), f32)]
        + [pltpu.VMEM((CH, EMB), f32)] * NBUF
        + [pltpu.SemaphoreType.DMA] * (2 * NBUF)
    )

    @functools.partial(pl.kernel, out_type=out_type, mesh=mesh,
                       scratch_types=scratch_types)
    def k(users_h, items_h, neg_h, Umlp_h, Umf_h, Ug1_h, Ug2_h, Vmlp_h, Vmf_h,
          umlp_o, umf_o, ug1_o, ug2_o, vmlp_o, vmf_o, negmlp_o, negmf_o,
          uidx, iidx, nidx, sbuf0, sbuf1, bbuf0, bbuf1, bbuf2, bbuf3, bbuf4,
          gsem0, gsem1, gsem2, gsem3, gsem4,
          wsem0, wsem1, wsem2, wsem3, wsem4):
        bufs = (bbuf0, bbuf1, bbuf2, bbuf3, bbuf4)
        gsems = (gsem0, gsem1, gsem2, gsem3, gsem4)
        wsems = (wsem0, wsem1, wsem2, wsem3, wsem4)
        sbufs = (sbuf0, sbuf1)
        wid = lax.axis_index("s") * NC + lax.axis_index("c")
        pltpu.sync_copy(users_h.at[pl.ds(wid * ub, ub)], uidx)
        pltpu.sync_copy(items_h.at[pl.ds(wid * ub, ub)], iidx)
        pltpu.sync_copy(neg_h.at[wid], nidx)

        small = [
            (Umlp_h, uidx, umlp_o), (Umf_h, uidx, umf_o),
            (Ug1_h, uidx, ug1_o), (Ug2_h, uidx, ug2_o),
            (Vmlp_h, iidx, vmlp_o), (Vmf_h, iidx, vmf_o),
        ]
        pend = [None, None]
        for n, (tbl, idx, out) in enumerate(small):
            s = n % 2
            if pend[s] is not None:
                cp, out_prev = pend[s]
                cp.wait()
                pltpu.sync_copy(sbufs[s], out_prev.at[pl.ds(wid * ub, ub)])
            pend[s] = (pltpu.async_copy(tbl.at[idx], sbufs[s], gsems[s]), out)
        for s in range(2):
            cp, out_prev = pend[s]
            cp.wait()
            pltpu.sync_copy(sbufs[s], out_prev.at[pl.ds(wid * ub, ub)])

        _neg_ring(wid, Vmlp_h, negmlp_o, nidx, nch, bufs, gsems, wsems)
        _neg_ring(wid, Vmf_h, negmf_o, nidx, nch, bufs, gsems, wsems)

    return k(users, items, neg_chunks, U_mlp, U_mf, U_mlp_g, U_mf_g,
             V_mlp, V_mf)


def _tc_pos(u_mlp, u_mf, v_mlp, v_mf, W1, b1r, W2, b2r):
    B = u_mlp.shape[0]
    f32 = jnp.float32

    def body(u_ref, umf_ref, v_ref, vmf_ref, W1_ref, b1_ref, W2_ref, b2_ref,
             mlp_o, mf_o, a_o):
        W1t = W1_ref[0:EMB, :]
        W1b = W1_ref[EMB:2 * EMB, :]
        A = jnp.dot(u_ref[...], W1t, preferred_element_type=f32) + b1_ref[0:1, :]
        a_o[...] = A
        hpos = jnp.maximum(
            A + jnp.dot(v_ref[...], W1b, preferred_element_type=f32), 0.0)
        mlp_o[...] = (jnp.dot(hpos, W2_ref[...], preferred_element_type=f32)
                      + b2_ref[0:1, :])
        mf_o[...] = umf_ref[...] * vmf_ref[...]

    full2 = lambda shape: pl.BlockSpec(shape, lambda: (0, 0))
    out_shape = [jax.ShapeDtypeStruct((B, EMB), f32)] * 3
    return pl.pallas_call(
        body,
        in_specs=[full2((B, EMB))] * 4 + [full2((2 * EMB, EMB)),
                                          full2((1, EMB)),
                                          full2((EMB, EMB)),
                                          full2((1, EMB))],
        out_specs=[full2((B, EMB))] * 3,
        out_shape=out_shape,
    )(u_mlp, u_mf, v_mlp, v_mf, W1, b1r, W2, b2r)


def _tc_negmlp(a_rows, neg_mlp_rows, W1, W2, b2r, nneg):
    B = a_rows.shape[0]
    NB = neg_mlp_rows.shape[0]
    f32 = jnp.float32

    def body(a_ref, nm_ref, W1_ref, W2_ref, b2_ref, negmlp_o):
        W1b = W1_ref[EMB:2 * EMB, :]
        M = jnp.dot(nm_ref[...], W1b, preferred_element_type=f32)
        H = jnp.maximum(a_ref[...] + M, 0.0)
        negmlp_o[...] = (jnp.dot(H, W2_ref[...], preferred_element_type=f32)
                         + b2_ref[0:1, :])

    res_spec = pl.BlockSpec((B, EMB), lambda i: (0, 0))
    blk_spec = pl.BlockSpec((B, EMB), lambda i: (i, 0))
    full = lambda shape: pl.BlockSpec(shape, lambda i: (0, 0))
    return pl.pallas_call(
        body,
        grid=(nneg,),
        in_specs=[res_spec, blk_spec, full((2 * EMB, EMB)),
                  full((EMB, EMB)), full((1, EMB))],
        out_specs=[blk_spec],
        out_shape=[jax.ShapeDtypeStruct((NB, EMB), f32)],
        compiler_params=pltpu.CompilerParams(
            dimension_semantics=("arbitrary",)),
    )(a_rows, neg_mlp_rows, W1, W2, b2r)[0]


def _tc_negmf(u_mf, neg_mf_rows, nneg):
    B = u_mf.shape[0]
    NB = neg_mf_rows.shape[0]
    f32 = jnp.float32

    def body(umf_ref, nf_ref, negmf_o):
        negmf_o[...] = umf_ref[...] * nf_ref[...]

    res_spec = pl.BlockSpec((B, EMB), lambda i: (0, 0))
    blk_spec = pl.BlockSpec((B, EMB), lambda i: (i, 0))
    return pl.pallas_call(
        body,
        grid=(nneg,),
        in_specs=[res_spec, blk_spec],
        out_specs=[blk_spec],
        out_shape=[jax.ShapeDtypeStruct((NB, EMB), f32)],
        compiler_params=pltpu.CompilerParams(
            dimension_semantics=("arbitrary",)),
    )(u_mf, neg_mf_rows)[0]


def kernel(users, items, neg_items, U_mlp, U_mf, V_mlp, V_mf,
           U_mlp_g, U_mf_g, W1, b1, W2, b2):
    B, NNEG = neg_items.shape
    i32 = jnp.int32
    users = users.astype(i32)
    items = items.astype(i32)
    nch = (B * NNEG) // (NW * CH)
    neg_chunks = jnp.swapaxes(neg_items.astype(i32), 0, 1).reshape(NW, nch, CH)

    (u_mlp, u_mf, u_mlp_g, u_mf_g, v_mlp, v_mf,
     neg_mlp_rows, neg_mf_rows) = _sc_gather_all(
        users, items, neg_chunks, U_mlp, U_mf, U_mlp_g, U_mf_g, V_mlp, V_mf)

    b1r = b1.reshape(1, EMB)
    b2r = b2.reshape(1, EMB)
    mlp_vector, mf_vector, a_rows = _tc_pos(
        u_mlp, u_mf, v_mlp, v_mf, W1, b1r, W2, b2r)
    negmlp_flat = _tc_negmlp(a_rows, neg_mlp_rows, W1, W2, b2r, NNEG)
    negmf_flat = _tc_negmf(u_mf, neg_mf_rows, NNEG)

    neg_mlp_vector = jnp.swapaxes(negmlp_flat.reshape(NNEG, B, EMB), 0, 1)
    neg_mf_vector = jnp.swapaxes(negmf_flat.reshape(NNEG, B, EMB), 0, 1)
    return (mlp_vector, mf_vector, u_mlp, u_mf, u_mlp_g, u_mf_g,
            neg_mlp_vector, neg_mf_vector)


# speedup vs baseline: 1.0731x; 1.0731x over previous
import functools

import jax
import jax.numpy as jnp
from jax import lax
from jax.experimental import pallas as pl
from jax.experimental.pallas import tpu as pltpu
from jax.experimental.pallas import tpu_sc as plsc

EMB = 128
NC = 2
NS = 16
NW = NC * NS
CH = 128
NBUF = 5


def _neg_ring(wid, tbl, out, nidx, nch, bufs, gsems, wsems):
    nb = nch * CH

    def body(g, carry):
        cps = []
        for j in range(NBUF):
            @pl.when(g > 0)
            def _(j=j):
                pltpu.make_async_copy(out.at[pl.ds(wid * nb, CH)],
                                      bufs[j], wsems[j]).wait()
            c = NBUF * g + j
            cps.append(pltpu.async_copy(tbl.at[nidx.at[c]], bufs[j],
                                        gsems[j]))
        for j in range(NBUF):
            cps[j].wait()
            c = NBUF * g + j
            pltpu.async_copy(bufs[j], out.at[pl.ds(wid * nb + c * CH, CH)],
                             wsems[j])
        return carry

    lax.fori_loop(0, nch // NBUF, body, 0)
    for j in range(NBUF):
        pltpu.make_async_copy(out.at[pl.ds(wid * nb, CH)], bufs[j],
                              wsems[j]).wait()


def _sc_gather_all(users, items, neg_chunks, U_mlp, U_mf, U_mlp_g, U_mf_g,
                   V_mlp, V_mf):
    B = users.shape[0]
    nch = neg_chunks.shape[1]
    NB = NW * nch * CH
    ub = B // NW

    mesh = plsc.VectorSubcoreMesh(core_axis_name="c", subcore_axis_name="s")
    f32 = jnp.float32
    out_type = (
        [jax.ShapeDtypeStruct((B, EMB), f32)] * 6
        + [jax.ShapeDtypeStruct((NB, EMB), f32)] * 2
    )
    scratch_types = (
        [pltpu.VMEM((ub,), jnp.int32),
         pltpu.VMEM((ub,), jnp.int32),
         pltpu.VMEM((nch, CH), jnp.int32),
         pltpu.VMEM((ub, EMB), f32),
         pltpu.VMEM((ub, EMB), f32)]
        + [pltpu.VMEM((CH, EMB), f32)] * NBUF
        + [pltpu.SemaphoreType.DMA] * (2 * NBUF)
    )

    @functools.partial(pl.kernel, out_type=out_type, mesh=mesh,
                       scratch_types=scratch_types)
    def k(users_h, items_h, neg_h, Umlp_h, Umf_h, Ug1_h, Ug2_h, Vmlp_h, Vmf_h,
          umlp_o, umf_o, ug1_o, ug2_o, vmlp_o, vmf_o, negmlp_o, negmf_o,
          uidx, iidx, nidx, sbuf0, sbuf1, bbuf0, bbuf1, bbuf2, bbuf3, bbuf4,
          gsem0, gsem1, gsem2, gsem3, gsem4,
          wsem0, wsem1, wsem2, wsem3, wsem4):
        bufs = (bbuf0, bbuf1, bbuf2, bbuf3, bbuf4)
        gsems = (gsem0, gsem1, gsem2, gsem3, gsem4)
        wsems = (wsem0, wsem1, wsem2, wsem3, wsem4)
        sbufs = (sbuf0, sbuf1)
        wid = lax.axis_index("s") * NC + lax.axis_index("c")
        pltpu.sync_copy(users_h.at[pl.ds(wid * ub, ub)], uidx)
        pltpu.sync_copy(items_h.at[pl.ds(wid * ub, ub)], iidx)
        pltpu.sync_copy(neg_h.at[wid], nidx)

        small = [
            (Umlp_h, uidx, umlp_o), (Umf_h, uidx, umf_o),
            (Ug1_h, uidx, ug1_o), (Ug2_h, uidx, ug2_o),
            (Vmlp_h, iidx, vmlp_o), (Vmf_h, iidx, vmf_o),
        ]
        pend = [None, None]
        for n, (tbl, idx, out) in enumerate(small):
            s = n % 2
            if pend[s] is not None:
                cp, out_prev = pend[s]
                cp.wait()
                pltpu.sync_copy(sbufs[s], out_prev.at[pl.ds(wid * ub, ub)])
            pend[s] = (pltpu.async_copy(tbl.at[idx], sbufs[s], gsems[s]), out)
        for s in range(2):
            cp, out_prev = pend[s]
            cp.wait()
            pltpu.sync_copy(sbufs[s], out_prev.at[pl.ds(wid * ub, ub)])

        _neg_ring(wid, Vmlp_h, negmlp_o, nidx, nch, bufs, gsems, wsems)
        _neg_ring(wid, Vmf_h, negmf_o, nidx, nch, bufs, gsems, wsems)

    return k(users, items, neg_chunks, U_mlp, U_mf, U_mlp_g, U_mf_g,
             V_mlp, V_mf)


def _tc_pos(u_mlp, u_mf, v_mlp, v_mf, W1, b1r, W2, b2r):
    B = u_mlp.shape[0]
    f32 = jnp.float32

    def body(u_ref, umf_ref, v_ref, vmf_ref, W1_ref, b1_ref, W2_ref, b2_ref,
             mlp_o, mf_o, a_o):
        W1t = W1_ref[0:EMB, :]
        W1b = W1_ref[EMB:2 * EMB, :]
        A = jnp.dot(u_ref[...], W1t, preferred_element_type=f32) + b1_ref[0:1, :]
        a_o[...] = A
        hpos = jnp.maximum(
            A + jnp.dot(v_ref[...], W1b, preferred_element_type=f32), 0.0)
        mlp_o[...] = (jnp.dot(hpos, W2_ref[...], preferred_element_type=f32)
                      + b2_ref[0:1, :])
        mf_o[...] = umf_ref[...] * vmf_ref[...]

    full2 = lambda shape: pl.BlockSpec(shape, lambda: (0, 0))
    out_shape = [jax.ShapeDtypeStruct((B, EMB), f32)] * 3
    return pl.pallas_call(
        body,
        in_specs=[full2((B, EMB))] * 4 + [full2((2 * EMB, EMB)),
                                          full2((1, EMB)),
                                          full2((EMB, EMB)),
                                          full2((1, EMB))],
        out_specs=[full2((B, EMB))] * 3,
        out_shape=out_shape,
    )(u_mlp, u_mf, v_mlp, v_mf, W1, b1r, W2, b2r)


def _tc_neg(a_rows, u_mf, neg_mlp_rows, neg_mf_rows, W1, W2, b2r, nneg):
    B = a_rows.shape[0]
    NB = neg_mlp_rows.shape[0]
    f32 = jnp.float32

    def body(a_ref, umf_ref, nm_ref, nf_ref, W1_ref, W2_ref, b2_ref,
             negmlp_o, negmf_o):
        W1b = W1_ref[EMB:2 * EMB, :]
        M = jnp.dot(nm_ref[...], W1b, preferred_element_type=f32)
        H = jnp.maximum(a_ref[...] + M, 0.0)
        negmlp_o[...] = (jnp.dot(H, W2_ref[...], preferred_element_type=f32)
                         + b2_ref[0:1, :])
        negmf_o[...] = umf_ref[...] * nf_ref[...]

    res_spec = pl.BlockSpec((B, EMB), lambda i: (0, 0))
    blk_spec = pl.BlockSpec((B, EMB), lambda i: (i, 0))
    full = lambda shape: pl.BlockSpec(shape, lambda i: (0, 0))
    out_shape = [jax.ShapeDtypeStruct((NB, EMB), f32)] * 2
    return pl.pallas_call(
        body,
        grid=(nneg,),
        in_specs=[res_spec, res_spec, blk_spec, blk_spec,
                  full((2 * EMB, EMB)), full((EMB, EMB)), full((1, EMB))],
        out_specs=[blk_spec, blk_spec],
        out_shape=out_shape,
        compiler_params=pltpu.CompilerParams(
            dimension_semantics=("arbitrary",)),
    )(a_rows, u_mf, neg_mlp_rows, neg_mf_rows, W1, W2, b2r)


def kernel(users, items, neg_items, U_mlp, U_mf, V_mlp, V_mf,
           U_mlp_g, U_mf_g, W1, b1, W2, b2):
    B, NNEG = neg_items.shape
    i32 = jnp.int32
    users = users.astype(i32)
    items = items.astype(i32)
    nch = (B * NNEG) // (NW * CH)
    neg_chunks = jnp.swapaxes(neg_items.astype(i32), 0, 1).reshape(NW, nch, CH)

    (u_mlp, u_mf, u_mlp_g, u_mf_g, v_mlp, v_mf,
     neg_mlp_rows, neg_mf_rows) = _sc_gather_all(
        users, items, neg_chunks, U_mlp, U_mf, U_mlp_g, U_mf_g, V_mlp, V_mf)

    b1r = b1.reshape(1, EMB)
    b2r = b2.reshape(1, EMB)
    mlp_vector, mf_vector, a_rows = _tc_pos(
        u_mlp, u_mf, v_mlp, v_mf, W1, b1r, W2, b2r)
    negmlp_flat, negmf_flat = _tc_neg(
        a_rows, u_mf, neg_mlp_rows, neg_mf_rows, W1, W2, b2r, NNEG)

    neg_mlp_vector = jnp.swapaxes(negmlp_flat.reshape(NNEG, B, EMB), 0, 1)
    neg_mf_vector = jnp.swapaxes(negmf_flat.reshape(NNEG, B, EMB), 0, 1)
    return (mlp_vector, mf_vector, u_mlp, u_mf, u_mlp_g, u_mf_g,
            neg_mlp_vector, neg_mf_vector)


# speedup vs baseline: 1.0845x; 1.0107x over previous
import functools

import jax
import jax.numpy as jnp
from jax import lax
from jax.experimental import pallas as pl
from jax.experimental.pallas import tpu as pltpu
from jax.experimental.pallas import tpu_sc as plsc

EMB = 128
NC = 2
NS = 16
NW = NC * NS
CH = 128
NBUF = 5


def _neg_ring(wid, tbl, out, nidx, nch, bufs, gsems, wsems):
    nb = nch * CH

    def body(g, carry):
        cps = []
        for j in range(NBUF):
            @pl.when(g > 0)
            def _(j=j):
                pltpu.make_async_copy(out.at[pl.ds(wid * nb, CH)],
                                      bufs[j], wsems[j]).wait()
            c = NBUF * g + j
            cps.append(pltpu.async_copy(tbl.at[nidx.at[c]], bufs[j],
                                        gsems[j]))
        for j in range(NBUF):
            cps[j].wait()
            c = NBUF * g + j
            pltpu.async_copy(bufs[j], out.at[pl.ds(wid * nb + c * CH, CH)],
                             wsems[j])
        return carry

    lax.fori_loop(0, nch // NBUF, body, 0)
    for j in range(NBUF):
        pltpu.make_async_copy(out.at[pl.ds(wid * nb, CH)], bufs[j],
                              wsems[j]).wait()


def _sc_gather_all(users, items, neg_chunks, U_mlp, U_mf, U_mlp_g, U_mf_g,
                   V_mlp, V_mf):
    B = users.shape[0]
    nch = neg_chunks.shape[1]
    NB = NW * nch * CH
    ub = B // NW

    mesh = plsc.VectorSubcoreMesh(core_axis_name="c", subcore_axis_name="s")
    f32 = jnp.float32
    out_type = (
        [jax.ShapeDtypeStruct((B, EMB), f32)] * 6
        + [jax.ShapeDtypeStruct((NB, EMB), f32)] * 2
    )
    scratch_types = (
        [pltpu.VMEM((ub,), jnp.int32),
         pltpu.VMEM((ub,), jnp.int32),
         pltpu.VMEM((nch, CH), jnp.int32),
         pltpu.VMEM((ub, EMB), f32),
         pltpu.VMEM((ub, EMB), f32)]
        + [pltpu.VMEM((CH, EMB), f32)] * NBUF
        + [pltpu.SemaphoreType.DMA] * (2 * NBUF)
    )

    @functools.partial(pl.kernel, out_type=out_type, mesh=mesh,
                       scratch_types=scratch_types)
    def k(users_h, items_h, neg_h, Umlp_h, Umf_h, Ug1_h, Ug2_h, Vmlp_h, Vmf_h,
          umlp_o, umf_o, ug1_o, ug2_o, vmlp_o, vmf_o, negmlp_o, negmf_o,
          uidx, iidx, nidx, sbuf0, sbuf1, bbuf0, bbuf1, bbuf2, bbuf3, bbuf4,
          gsem0, gsem1, gsem2, gsem3, gsem4,
          wsem0, wsem1, wsem2, wsem3, wsem4):
        bufs = (bbuf0, bbuf1, bbuf2, bbuf3, bbuf4)
        gsems = (gsem0, gsem1, gsem2, gsem3, gsem4)
        wsems = (wsem0, wsem1, wsem2, wsem3, wsem4)
        sbufs = (sbuf0, sbuf1)
        wid = lax.axis_index("s") * NC + lax.axis_index("c")
        pltpu.sync_copy(users_h.at[pl.ds(wid * ub, ub)], uidx)
        pltpu.sync_copy(items_h.at[pl.ds(wid * ub, ub)], iidx)
        pltpu.sync_copy(neg_h.at[wid], nidx)

        small = [
            (Umlp_h, uidx, umlp_o), (Umf_h, uidx, umf_o),
            (Ug1_h, uidx, ug1_o), (Ug2_h, uidx, ug2_o),
            (Vmlp_h, iidx, vmlp_o), (Vmf_h, iidx, vmf_o),
        ]
        pend = [None, None]
        for n, (tbl, idx, out) in enumerate(small):
            s = n % 2
            if pend[s] is not None:
                cp, out_prev = pend[s]
                cp.wait()
                pltpu.sync_copy(sbufs[s], out_prev.at[pl.ds(wid * ub, ub)])
            pend[s] = (pltpu.async_copy(tbl.at[idx], sbufs[s], gsems[s]), out)
        for s in range(2):
            cp, out_prev = pend[s]
            cp.wait()
            pltpu.sync_copy(sbufs[s], out_prev.at[pl.ds(wid * ub, ub)])

        _neg_ring(wid, Vmlp_h, negmlp_o, nidx, nch, bufs, gsems, wsems)
        _neg_ring(wid, Vmf_h, negmf_o, nidx, nch, bufs, gsems, wsems)

    return k(users, items, neg_chunks, U_mlp, U_mf, U_mlp_g, U_mf_g,
             V_mlp, V_mf)


def _tc_head(u_mlp, u_mf, v_mlp, v_mf, neg_mlp_rows, neg_mf_rows,
             W1, b1r, W2, b2r, nneg):
    B = u_mlp.shape[0]
    NB = neg_mlp_rows.shape[0]
    f32 = jnp.float32

    def body(u_ref, umf_ref, v_ref, vmf_ref, nm_ref, nf_ref,
             W1_ref, b1_ref, W2_ref, b2_ref,
             mlp_o, mf_o, negmlp_o, negmf_o, a_scr):
        W1b = W1_ref[EMB:2 * EMB, :]
        W2 = W2_ref[...]
        b2 = b2_ref[0:1, :]

        @pl.when(pl.program_id(0) == 0)
        def _():
            W1t = W1_ref[0:EMB, :]
            A = (jnp.dot(u_ref[...], W1t, preferred_element_type=f32)
                 + b1_ref[0:1, :])
            a_scr[...] = A
            hpos = jnp.maximum(
                A + jnp.dot(v_ref[...], W1b, preferred_element_type=f32), 0.0)
            mlp_o[...] = jnp.dot(hpos, W2, preferred_element_type=f32) + b2
            mf_o[...] = umf_ref[...] * vmf_ref[...]

        M = jnp.dot(nm_ref[...], W1b, preferred_element_type=f32)
        H = jnp.maximum(a_scr[...] + M, 0.0)
        negmlp_o[...] = jnp.dot(H, W2, preferred_element_type=f32) + b2
        negmf_o[...] = umf_ref[...] * nf_ref[...]

    res_spec = pl.BlockSpec((B, EMB), lambda i: (0, 0))
    blk_spec = pl.BlockSpec((B, EMB), lambda i: (i, 0))
    full = lambda shape: pl.BlockSpec(shape, lambda i: (0, 0))
    return pl.pallas_call(
        body,
        grid=(nneg,),
        in_specs=[res_spec, res_spec, res_spec, res_spec, blk_spec, blk_spec,
                  full((2 * EMB, EMB)), full((1, EMB)),
                  full((EMB, EMB)), full((1, EMB))],
        out_specs=[res_spec, res_spec, blk_spec, blk_spec],
        out_shape=[jax.ShapeDtypeStruct((B, EMB), f32)] * 2
        + [jax.ShapeDtypeStruct((NB, EMB), f32)] * 2,
        scratch_shapes=[pltpu.VMEM((B, EMB), f32)],
        compiler_params=pltpu.CompilerParams(
            dimension_semantics=("arbitrary",)),
    )(u_mlp, u_mf, v_mlp, v_mf, neg_mlp_rows, neg_mf_rows, W1, b1r, W2, b2r)


def kernel(users, items, neg_items, U_mlp, U_mf, V_mlp, V_mf,
           U_mlp_g, U_mf_g, W1, b1, W2, b2):
    B, NNEG = neg_items.shape
    i32 = jnp.int32
    users = users.astype(i32)
    items = items.astype(i32)
    nch = (B * NNEG) // (NW * CH)
    neg_chunks = jnp.swapaxes(neg_items.astype(i32), 0, 1).reshape(NW, nch, CH)

    (u_mlp, u_mf, u_mlp_g, u_mf_g, v_mlp, v_mf,
     neg_mlp_rows, neg_mf_rows) = _sc_gather_all(
        users, items, neg_chunks, U_mlp, U_mf, U_mlp_g, U_mf_g, V_mlp, V_mf)

    b1r = b1.reshape(1, EMB)
    b2r = b2.reshape(1, EMB)
    mlp_vector, mf_vector, negmlp_flat, negmf_flat = _tc_head(
        u_mlp, u_mf, v_mlp, v_mf, neg_mlp_rows, neg_mf_rows,
        W1, b1r, W2, b2r, NNEG)

    neg_mlp_vector = jnp.swapaxes(negmlp_flat.reshape(NNEG, B, EMB), 0, 1)
    neg_mf_vector = jnp.swapaxes(negmf_flat.reshape(NNEG, B, EMB), 0, 1)
    return (mlp_vector, mf_vector, u_mlp, u_mf, u_mlp_g, u_mf_g,
            neg_mlp_vector, neg_mf_vector)
